# Initial kernel scaffold; baseline (speedup 1.0000x reference)
import jax
import jax.numpy as jnp
from jax.experimental import pallas as pl


def kernel(x):
    raise NotImplementedError("write your pallas kernel here")


# speedup vs baseline: 1.0284x; 1.0284x over previous
import functools

import jax
import jax.numpy as jnp
from jax.experimental import pallas as pl
from jax.experimental.pallas import tpu as pltpu

_LANES = 128
_EPS = 1e-5
_VMEM_LIMIT_BYTES = 64 * 1024 * 1024


def _rms_body(x_ref, o_ref, *, n: int, inv_n: float, eps: float):
    x = x_ref[...]
    x2 = x * x
    lane = jax.lax.broadcasted_iota(jnp.int32, x2.shape, 1)
    lo = lane < n
    s_all = jnp.sum(x2, axis=-1, keepdims=True)
    s_lo = jnp.sum(jnp.where(lo, x2, 0.0), axis=-1, keepdims=True)
    s_hi = s_all - s_lo
    inv_lo = jax.lax.rsqrt(s_lo * inv_n + eps)
    inv_hi = jax.lax.rsqrt(s_hi * inv_n + eps)
    o_ref[...] = x * jnp.where(lo, inv_lo, inv_hi)


def kernel(x):
    batch, features, dim1, dim2 = x.shape
    n = features
    total = batch * features * dim1 * dim2
    assert n < _LANES and _LANES % n == 0 and total % _LANES == 0

    rows = total // _LANES
    x_lane = jnp.reshape(x, (rows, _LANES))

    blk = 8192
    while rows % blk != 0:
        blk //= 2

    body = functools.partial(_rms_body, n=n, inv_n=1.0 / float(n), eps=_EPS)
    out = pl.pallas_call(
        body,
        out_shape=jax.ShapeDtypeStruct((rows, _LANES), x.dtype),
        grid_spec=pl.GridSpec(
            grid=(rows // blk,),
            in_specs=[pl.BlockSpec((blk, _LANES), lambda i: (i, 0))],
            out_specs=pl.BlockSpec((blk, _LANES), lambda i: (i, 0)),
        ),
        compiler_params=pltpu.CompilerParams(
            dimension_semantics=("parallel",),
            vmem_limit_bytes=_VMEM_LIMIT_BYTES,
        ),
    )(x_lane)
    return jnp.reshape(out, x.shape)


# speedup vs baseline: 1.5081x; 1.4664x over previous
import functools

import jax
import jax.numpy as jnp
from jax.experimental import pallas as pl
from jax.experimental.pallas import tpu as pltpu

_EPS = 1e-5
_VMEM_LIMIT_BYTES = 64 * 1024 * 1024


def _rms_body(x_ref, o_ref, *, inv_n: float, eps: float):
    x = x_ref[...]
    x2 = x * x
    s = jnp.sum(x2, axis=-1, keepdims=True)
    s_down = jnp.roll(s, -1, axis=1)
    s_up = jnp.roll(s, 1, axis=1)
    h = jax.lax.broadcasted_iota(jnp.int32, s.shape, 1)
    pair = s + jnp.where(h % 2 == 0, s_down, s_up)
    o_ref[...] = x * jax.lax.rsqrt(pair * inv_n + eps)


def kernel(x):
    batch, features, dim1, dim2 = x.shape
    n = features
    assert n == 2 * dim2, "group = 2 consecutive rows of the last dim"

    rows = batch * features
    x3d = jnp.reshape(x, (rows, dim1, dim2))

    blk = 256
    while rows % blk != 0:
        blk //= 2

    body = functools.partial(_rms_body, inv_n=1.0 / float(n), eps=_EPS)
    out = pl.pallas_call(
        body,
        out_shape=jax.ShapeDtypeStruct((rows, dim1, dim2), x.dtype),
        grid_spec=pl.GridSpec(
            grid=(rows // blk,),
            in_specs=[pl.BlockSpec((blk, dim1, dim2), lambda i: (i, 0, 0))],
            out_specs=pl.BlockSpec((blk, dim1, dim2), lambda i: (i, 0, 0)),
        ),
        compiler_params=pltpu.CompilerParams(
            dimension_semantics=("parallel",),
            vmem_limit_bytes=_VMEM_LIMIT_BYTES,
        ),
    )(x3d)
    return jnp.reshape(out, x.shape)
